# baseline (speedup 1.0000x reference)
import functools

import jax
import jax.numpy as jnp
from jax import lax
from jax.experimental import pallas as pl
from jax.experimental.pallas import tpu as pltpu
from jax.experimental.pallas import tpu_sc as plsc

EMBED_DIM = 64
CHUNK = 128


def _make_sc_gather(n_chunks_per_worker, nc, ns):
    nw = nc * ns
    rows_per_worker = n_chunks_per_worker * CHUNK
    mesh = plsc.VectorSubcoreMesh(core_axis_name="c", subcore_axis_name="s")

    @functools.partial(
        pl.kernel,
        mesh=mesh,
        compiler_params=pltpu.CompilerParams(use_tc_tiling_on_sc=False),
        out_type=jax.ShapeDtypeStruct(
            (nw * rows_per_worker, EMBED_DIM), jnp.float32
        ),
        scratch_types=[
            pltpu.VMEM((n_chunks_per_worker, CHUNK), jnp.int32),
            pltpu.VMEM((rows_per_worker, EMBED_DIM), jnp.float32),
            pltpu.SemaphoreType.DMA,
        ],
    )
    def sc_gather(idx_hbm, table_hbm, out_hbm, idx_v, rows_v, sem):
        wid = lax.axis_index("s") * nc + lax.axis_index("c")
        pltpu.sync_copy(idx_hbm.at[wid], idx_v)
        copies = []
        for j in range(n_chunks_per_worker):
            copies.append(
                pltpu.async_copy(
                    table_hbm.at[idx_v.at[j]],
                    rows_v.at[pl.ds(j * CHUNK, CHUNK)],
                    sem,
                )
            )
        for c in copies:
            c.wait()
        pltpu.sync_copy(
            rows_v, out_hbm.at[pl.ds(wid * rows_per_worker, rows_per_worker)]
        )

    return sc_gather


def _ff_softmax_body(x_ref, w2_ref, b2_ref, wout_ref, bout_ref, o_ref):
    x = x_ref[...]
    h = jnp.maximum(
        jnp.dot(x, w2_ref[...], preferred_element_type=jnp.float32)
        + b2_ref[...],
        0.0,
    )
    logits = (
        jnp.dot(h, wout_ref[...], preferred_element_type=jnp.float32)
        + bout_ref[...]
    )
    m = jnp.max(logits, axis=-1, keepdims=True)
    e = jnp.exp(logits - m)
    o_ref[...] = e / jnp.sum(e, axis=-1, keepdims=True)


def kernel(input_seq, output_seq, emb, W1, b1, W2, b2, Wout, bout):
    del input_seq, W1, b1

    batch, out_len = output_seq.shape
    n_rows = batch * out_len
    hidden = W2.shape[1]
    out_vocab = Wout.shape[1]

    info = plsc.get_sparse_core_info()
    nc, ns = info.num_cores, info.num_subcores
    nw = nc * ns

    idx = output_seq.reshape(-1).astype(jnp.int32)
    per_worker = -(-n_rows // (nw * CHUNK)) * CHUNK
    n_pad = nw * per_worker
    idx = jnp.pad(idx, (0, n_pad - n_rows))
    idx = idx.reshape(nw, per_worker // CHUNK, CHUNK)

    gathered = _make_sc_gather(per_worker // CHUNK, nc, ns)(idx, emb)

    tile_rows = 512
    grid = n_rows // tile_rows
    out = pl.pallas_call(
        _ff_softmax_body,
        grid=(grid,),
        in_specs=[
            pl.BlockSpec((tile_rows, EMBED_DIM), lambda i: (i, 0)),
            pl.BlockSpec((EMBED_DIM, hidden), lambda i: (0, 0)),
            pl.BlockSpec((1, hidden), lambda i: (0, 0)),
            pl.BlockSpec((hidden, out_vocab), lambda i: (0, 0)),
            pl.BlockSpec((1, out_vocab), lambda i: (0, 0)),
        ],
        out_specs=pl.BlockSpec((tile_rows, out_vocab), lambda i: (i, 0)),
        out_shape=jax.ShapeDtypeStruct((n_rows, out_vocab), jnp.float32),
    )(
        gathered,
        W2,
        b2.reshape(1, hidden),
        Wout,
        bout.reshape(1, out_vocab),
    )

    return out.reshape(batch, out_len, out_vocab)
